# Initial kernel scaffold; baseline (speedup 1.0000x reference)
import jax
import jax.numpy as jnp
from jax.experimental import pallas as pl


def kernel(hidden_states, expert_indices, expert_weights, up_adapters, gate_adapters, up_mixture_logits, gate_mixture_logits, down_projections, up_bank, gate_bank):
    raise NotImplementedError("write your pallas kernel here")


# speedup vs baseline: 2.1542x; 2.1542x over previous
import functools

import jax
import jax.numpy as jnp
from jax.experimental import pallas as pl
from jax.experimental.pallas import tpu as pltpu

E = 8
TOKEN_BLOCK = 256


def _mix_body(up_logits_ref, gate_logits_ref, up_bank_ref, gate_bank_ref,
              up_out_ref, gate_out_ref):
    for lref, bref, oref in ((up_logits_ref, up_bank_ref, up_out_ref),
                             (gate_logits_ref, gate_bank_ref, gate_out_ref)):
        logits = lref[...]
        m = jnp.max(logits, axis=1, keepdims=True)
        ex = jnp.exp(logits - m)
        alpha = ex / jnp.sum(ex, axis=1, keepdims=True)
        mixed = jax.lax.dot_general(
            alpha, bref[...], (((1,), (0,)), ((), ())),
            preferred_element_type=jnp.float32)
        oref[...] = mixed.astype(jnp.bfloat16)


def _mix_banks(up_mixture_logits, gate_mixture_logits, up_bank, gate_bank):
    ku, d, r = up_bank.shape
    kg = gate_bank.shape[0]
    up_flat = up_bank.reshape(ku, d * r)
    gate_flat = gate_bank.reshape(kg, d * r)
    up_mixed, gate_mixed = pl.pallas_call(
        _mix_body,
        out_shape=(jax.ShapeDtypeStruct((E, d * r), jnp.bfloat16),
                   jax.ShapeDtypeStruct((E, d * r), jnp.bfloat16)),
    )(up_mixture_logits, gate_mixture_logits, up_flat, gate_flat)
    return up_mixed.reshape(E, d, r), gate_mixed.reshape(E, d, r)


def _ffn_body(x_ref, idx_ref, w_ref, upm_ref, gm_ref, ua_ref, ga_ref, dn_ref,
              out_ref, acc_ref):
    e = pl.program_id(0)
    t = pl.program_id(1)
    x = x_ref[...]
    f32 = jnp.float32
    tb_up = jax.lax.dot_general(
        x, upm_ref[0], (((1,), (0,)), ((), ())), preferred_element_type=f32)
    tb_gate = jax.lax.dot_general(
        x, gm_ref[0], (((1,), (0,)), ((), ())), preferred_element_type=f32)
    up = jax.lax.dot_general(
        tb_up.astype(jnp.bfloat16), ua_ref[0], (((1,), (1,)), ((), ())),
        preferred_element_type=f32)
    gate = jax.lax.dot_general(
        tb_gate.astype(jnp.bfloat16), ga_ref[0], (((1,), (1,)), ((), ())),
        preferred_element_type=f32)
    inter = (gate / (1.0 + jnp.exp(-gate))) * up
    y = jax.lax.dot_general(
        inter.astype(jnp.bfloat16), dn_ref[0], (((1,), (1,)), ((), ())),
        preferred_element_type=f32)
    coef = jnp.sum(jnp.where(idx_ref[...] == e, w_ref[...], 0.0), axis=1,
                   keepdims=True)
    contrib = coef * y
    sl = pl.ds(t * TOKEN_BLOCK, TOKEN_BLOCK)

    @pl.when(e == 0)
    def _():
        acc_ref[sl, :] = contrib

    @pl.when(e != 0)
    def _():
        acc_ref[sl, :] = acc_ref[sl, :] + contrib

    @pl.when(e == pl.num_programs(0) - 1)
    def _():
        out_ref[...] = acc_ref[sl, :]


def kernel(hidden_states, expert_indices, expert_weights, up_adapters,
           gate_adapters, up_mixture_logits, gate_mixture_logits,
           down_projections, up_bank, gate_bank):
    seq, d = hidden_states.shape
    p, r = up_adapters.shape[1:]
    nb = seq // TOKEN_BLOCK

    up_mixed, gate_mixed = _mix_banks(up_mixture_logits, gate_mixture_logits,
                                      up_bank, gate_bank)
    x16 = hidden_states.astype(jnp.bfloat16)
    ua16 = up_adapters.astype(jnp.bfloat16)
    ga16 = gate_adapters.astype(jnp.bfloat16)
    dn16 = down_projections.astype(jnp.bfloat16)
    idx = expert_indices.astype(jnp.int32)

    out = pl.pallas_call(
        _ffn_body,
        grid=(E, nb),
        in_specs=[
            pl.BlockSpec((TOKEN_BLOCK, d), lambda e, t: (t, 0)),
            pl.BlockSpec((TOKEN_BLOCK, 2), lambda e, t: (t, 0)),
            pl.BlockSpec((TOKEN_BLOCK, 2), lambda e, t: (t, 0)),
            pl.BlockSpec((1, d, r), lambda e, t: (e, 0, 0)),
            pl.BlockSpec((1, d, r), lambda e, t: (e, 0, 0)),
            pl.BlockSpec((1, p, r), lambda e, t: (e, 0, 0)),
            pl.BlockSpec((1, p, r), lambda e, t: (e, 0, 0)),
            pl.BlockSpec((1, d, p), lambda e, t: (e, 0, 0)),
        ],
        out_specs=pl.BlockSpec((TOKEN_BLOCK, d), lambda e, t: (t, 0)),
        out_shape=jax.ShapeDtypeStruct((seq, d), jnp.float32),
        scratch_shapes=[pltpu.VMEM((seq, d), jnp.float32)],
    )(x16, idx, expert_weights, up_mixed, gate_mixed, ua16, ga16, dn16)
    return out


# speedup vs baseline: 2.2202x; 1.0307x over previous
import functools

import jax
import jax.numpy as jnp
from jax.experimental import pallas as pl
from jax.experimental.pallas import tpu as pltpu

E = 8
TOKEN_BLOCK = 256


def _mix_body(up_logits_ref, gate_logits_ref, up_bank_ref, gate_bank_ref,
              up_out_ref, gate_out_ref):
    for lref, bref, oref in ((up_logits_ref, up_bank_ref, up_out_ref),
                             (gate_logits_ref, gate_bank_ref, gate_out_ref)):
        logits = lref[...]
        m = jnp.max(logits, axis=1, keepdims=True)
        ex = jnp.exp(logits - m)
        alpha = ex / jnp.sum(ex, axis=1, keepdims=True)
        mixed = jax.lax.dot_general(
            alpha, bref[...], (((1,), (0,)), ((), ())),
            preferred_element_type=jnp.float32)
        oref[...] = mixed.astype(jnp.bfloat16)


def _mix_banks(up_mixture_logits, gate_mixture_logits, up_bank, gate_bank):
    ku, d, r = up_bank.shape
    kg = gate_bank.shape[0]
    up_flat = up_bank.reshape(ku, d * r)
    gate_flat = gate_bank.reshape(kg, d * r)
    up_mixed, gate_mixed = pl.pallas_call(
        _mix_body,
        out_shape=(jax.ShapeDtypeStruct((E, d * r), jnp.bfloat16),
                   jax.ShapeDtypeStruct((E, d * r), jnp.bfloat16)),
    )(up_mixture_logits, gate_mixture_logits, up_flat, gate_flat)
    return up_mixed.reshape(E, d, r), gate_mixed.reshape(E, d, r)


def _ffn_body(x_ref, idx0_ref, idx1_ref, w0_ref, w1_ref, upm_ref, gm_ref,
              ua_ref, ga_ref, dn_ref, out_ref, acc_ref):
    e = pl.program_id(0)
    t = pl.program_id(1)
    x = x_ref[...]
    f32 = jnp.float32
    coef = (jnp.where(idx0_ref[...] == e, w0_ref[...], 0.0)
            + jnp.where(idx1_ref[...] == e, w1_ref[...], 0.0))
    tb_up = jax.lax.dot_general(
        x, upm_ref[0], (((1,), (0,)), ((), ())), preferred_element_type=f32)
    tb_gate = jax.lax.dot_general(
        x, gm_ref[0], (((1,), (0,)), ((), ())), preferred_element_type=f32)
    up = jax.lax.dot_general(
        (coef * tb_up).astype(jnp.bfloat16), ua_ref[0],
        (((1,), (1,)), ((), ())), preferred_element_type=f32)
    gate = jax.lax.dot_general(
        tb_gate.astype(jnp.bfloat16), ga_ref[0], (((1,), (1,)), ((), ())),
        preferred_element_type=f32)
    inter = gate * (0.5 + 0.5 * jnp.tanh(0.5 * gate)) * up
    y = jax.lax.dot_general(
        inter.astype(jnp.bfloat16), dn_ref[0], (((1,), (1,)), ((), ())),
        preferred_element_type=f32)
    sl = pl.ds(t * TOKEN_BLOCK, TOKEN_BLOCK)

    @pl.when(e == 0)
    def _():
        acc_ref[sl, :] = y

    @pl.when(e != 0)
    def _():
        acc_ref[sl, :] = acc_ref[sl, :] + y

    @pl.when(e == pl.num_programs(0) - 1)
    def _():
        out_ref[...] = acc_ref[sl, :]


def kernel(hidden_states, expert_indices, expert_weights, up_adapters,
           gate_adapters, up_mixture_logits, gate_mixture_logits,
           down_projections, up_bank, gate_bank):
    seq, d = hidden_states.shape
    p, r = up_adapters.shape[1:]
    nb = seq // TOKEN_BLOCK

    up_mixed, gate_mixed = _mix_banks(up_mixture_logits, gate_mixture_logits,
                                      up_bank, gate_bank)
    x16 = hidden_states.astype(jnp.bfloat16)
    ua16 = up_adapters.astype(jnp.bfloat16)
    ga16 = gate_adapters.astype(jnp.bfloat16)
    dn16 = down_projections.astype(jnp.bfloat16)
    idx = expert_indices.astype(jnp.int32)
    idx0, idx1 = idx[:, 0:1], idx[:, 1:2]
    w0, w1 = expert_weights[:, 0:1], expert_weights[:, 1:2]

    out = pl.pallas_call(
        _ffn_body,
        grid=(E, nb),
        in_specs=[
            pl.BlockSpec((TOKEN_BLOCK, d), lambda e, t: (t, 0)),
            pl.BlockSpec((TOKEN_BLOCK, 1), lambda e, t: (t, 0)),
            pl.BlockSpec((TOKEN_BLOCK, 1), lambda e, t: (t, 0)),
            pl.BlockSpec((TOKEN_BLOCK, 1), lambda e, t: (t, 0)),
            pl.BlockSpec((TOKEN_BLOCK, 1), lambda e, t: (t, 0)),
            pl.BlockSpec((1, d, r), lambda e, t: (e, 0, 0)),
            pl.BlockSpec((1, d, r), lambda e, t: (e, 0, 0)),
            pl.BlockSpec((1, p, r), lambda e, t: (e, 0, 0)),
            pl.BlockSpec((1, p, r), lambda e, t: (e, 0, 0)),
            pl.BlockSpec((1, d, p), lambda e, t: (e, 0, 0)),
        ],
        out_specs=pl.BlockSpec((TOKEN_BLOCK, d), lambda e, t: (t, 0)),
        out_shape=jax.ShapeDtypeStruct((seq, d), jnp.float32),
        scratch_shapes=[pltpu.VMEM((seq, d), jnp.float32)],
    )(x16, idx0, idx1, w0, w1, up_mixed, gate_mixed, ua16, ga16, dn16)
    return out


# speedup vs baseline: 2.7103x; 1.2207x over previous
import functools

import jax
import jax.numpy as jnp
from jax import lax
from jax.experimental import pallas as pl
from jax.experimental.pallas import tpu as pltpu

E = 8
TOPK = 2
BLK = 256


def _mix_body(up_logits_ref, gate_logits_ref, up_bank_ref, gate_bank_ref,
              up_out_ref, gate_out_ref):
    for lref, bref, oref in ((up_logits_ref, up_bank_ref, up_out_ref),
                             (gate_logits_ref, gate_bank_ref, gate_out_ref)):
        logits = lref[...]
        m = jnp.max(logits, axis=1, keepdims=True)
        ex = jnp.exp(logits - m)
        alpha = ex / jnp.sum(ex, axis=1, keepdims=True)
        mixed = jax.lax.dot_general(
            alpha, bref[...], (((1,), (0,)), ((), ())),
            preferred_element_type=jnp.float32)
        oref[...] = mixed.astype(jnp.bfloat16)


def _mix_banks(up_mixture_logits, gate_mixture_logits, up_bank, gate_bank):
    ku, d, r = up_bank.shape
    kg = gate_bank.shape[0]
    up_flat = up_bank.reshape(ku, d * r)
    gate_flat = gate_bank.reshape(kg, d * r)
    up_mixed, gate_mixed = pl.pallas_call(
        _mix_body,
        out_shape=(jax.ShapeDtypeStruct((E, d * r), jnp.bfloat16),
                   jax.ShapeDtypeStruct((E, d * r), jnp.bfloat16)),
    )(up_mixture_logits, gate_mixture_logits, up_flat, gate_flat)
    return up_mixed.reshape(E, d, r), gate_mixed.reshape(E, d, r)


def _routing_metadata(expert_indices, expert_weights, nb):
    i32 = jnp.int32
    g = expert_indices.size
    flat_e = expert_indices.reshape(-1).astype(i32)
    flat_w = expert_weights.reshape(-1)
    oh = flat_e[:, None] == jnp.arange(E, dtype=i32)[None, :]
    ohf = oh.astype(i32)
    ranks_all = jnp.cumsum(ohf, axis=0) - ohf
    rank = jnp.sum(jnp.where(oh, ranks_all, 0), axis=1)
    counts = jnp.sum(ohf, axis=0)
    bpe = (counts + BLK - 1) // BLK
    cumb = jnp.cumsum(bpe)
    base_slot = (jnp.concatenate([jnp.zeros(1, i32), cumb[:-1]]) * BLK)
    slot = base_slot[flat_e] + rank
    used = cumb[-1]
    bids = jnp.arange(nb, dtype=i32)
    raw_be = jnp.searchsorted(cumb, bids, side="right").astype(i32)
    last_e = jnp.searchsorted(cumb, used - 1, side="right").astype(i32)
    be = jnp.minimum(raw_be, last_e)
    block_start = bids * BLK
    nv = jnp.clip(counts[be] - (block_start - base_slot[be]), 0, BLK)
    nv = nv.astype(i32)
    first = ((block_start == base_slot[be]) & (nv > 0)).astype(i32)
    return flat_w, slot, be, nv, first


def _ffn_grouped_body(be_ref, nv_ref, first_ref, x_ref, sw_ref, upm_ref,
                      gm_ref, ua_ref, ga_ref, dn_ref, y_ref, dnbf_ref):
    i = pl.program_id(0)
    f32 = jnp.float32
    bf16 = jnp.bfloat16

    @pl.when(first_ref[i] == 1)
    def _():
        dnbf_ref[...] = dn_ref[0].astype(bf16)

    @pl.when(nv_ref[i] > 0)
    def _():
        x = x_ref[...].astype(bf16)
        coef = sw_ref[...]
        tb_up = jax.lax.dot_general(
            x, upm_ref[0], (((1,), (0,)), ((), ())),
            preferred_element_type=f32)
        tb_gate = jax.lax.dot_general(
            x, gm_ref[0], (((1,), (0,)), ((), ())),
            preferred_element_type=f32)
        up = jax.lax.dot_general(
            (coef * tb_up).astype(bf16), ua_ref[0],
            (((1,), (1,)), ((), ())), preferred_element_type=f32)
        gate = jax.lax.dot_general(
            tb_gate.astype(bf16), ga_ref[0],
            (((1,), (1,)), ((), ())), preferred_element_type=f32)
        inter = gate * (0.5 + 0.5 * jnp.tanh(0.5 * gate)) * up
        y_ref[...] = jax.lax.dot_general(
            inter.astype(bf16), dnbf_ref[...],
            (((1,), (1,)), ((), ())), preferred_element_type=f32)


def _ffn_grouped(x_sorted, slot_w, up_mixed, gate_mixed, up_adapters,
                 gate_adapters, down_projections, be, nv, first, nb):
    nbb, d = x_sorted.shape
    p, r = up_adapters.shape[1:]
    grid_spec = pltpu.PrefetchScalarGridSpec(
        num_scalar_prefetch=3,
        grid=(nb,),
        in_specs=[
            pl.BlockSpec((BLK, d), lambda i, be, nv, fs: (i, 0)),
            pl.BlockSpec((BLK, 1), lambda i, be, nv, fs: (i, 0)),
            pl.BlockSpec((1, d, r), lambda i, be, nv, fs: (be[i], 0, 0)),
            pl.BlockSpec((1, d, r), lambda i, be, nv, fs: (be[i], 0, 0)),
            pl.BlockSpec((1, p, r), lambda i, be, nv, fs: (be[i], 0, 0)),
            pl.BlockSpec((1, p, r), lambda i, be, nv, fs: (be[i], 0, 0)),
            pl.BlockSpec((1, d, p), lambda i, be, nv, fs: (be[i], 0, 0)),
        ],
        out_specs=pl.BlockSpec((BLK, d), lambda i, be, nv, fs: (i, 0)),
        scratch_shapes=[
            pltpu.VMEM((d, p), jnp.bfloat16),
        ],
    )
    return pl.pallas_call(
        _ffn_grouped_body,
        grid_spec=grid_spec,
        out_shape=jax.ShapeDtypeStruct((nbb, d), jnp.float32),
    )(be, nv, first, x_sorted, slot_w.reshape(nbb, 1), up_mixed, gate_mixed,
      up_adapters.astype(jnp.bfloat16), gate_adapters.astype(jnp.bfloat16),
      down_projections)


def kernel(hidden_states, expert_indices, expert_weights, up_adapters,
           gate_adapters, up_mixture_logits, gate_mixture_logits,
           down_projections, up_bank, gate_bank):
    seq, d = hidden_states.shape
    g = seq * TOPK
    nb = g // BLK + E
    nbb = nb * BLK

    up_mixed, gate_mixed = _mix_banks(up_mixture_logits, gate_mixture_logits,
                                      up_bank, gate_bank)
    flat_w, slot, be, nv, first = _routing_metadata(
        expert_indices, expert_weights, nb)

    flat_t = jnp.arange(g, dtype=jnp.int32) // TOPK
    slot_token = jnp.zeros((nbb,), jnp.int32).at[slot].set(flat_t)
    slot_w = jnp.zeros((nbb,), jnp.float32).at[slot].set(flat_w)
    x_sorted = jnp.take(hidden_states, slot_token, axis=0)

    y_sorted = _ffn_grouped(x_sorted, slot_w, up_mixed, gate_mixed,
                            up_adapters, gate_adapters, down_projections,
                            be, nv, first, nb)

    pos = slot.reshape(seq, TOPK)
    out = (jnp.take(y_sorted, pos[:, 0], axis=0)
           + jnp.take(y_sorted, pos[:, 1], axis=0))
    return out
